# Initial kernel scaffold; baseline (speedup 1.0000x reference)
import jax
import jax.numpy as jnp
from jax.experimental import pallas as pl


def kernel(x, pos_edge_index, neg_edge_index, W1, b1, W2, b2, W3, b3):
    raise NotImplementedError("write your pallas kernel here")


# speedup vs baseline: 10.7589x; 10.7589x over previous
import functools

import jax
import jax.numpy as jnp
from jax import lax
from jax.experimental import pallas as pl
from jax.experimental.pallas import tpu as pltpu
from jax.experimental.pallas import tpu_sc as plsc

N = 10000
E = 320000
IN_C = 128
HID = 128
OUT_C = 64

NC = 2
NS = 16
NW = NC * NS
EPW = E // NW
CH = 80
NCHUNK = EPW // CH
NP = 10240
NPAD = 10240
SLC = NP // NS
RPS = NPAD // NS


def _mesh():
    return plsc.VectorSubcoreMesh(
        core_axis_name="c", subcore_axis_name="s",
        num_cores=NC, num_subcores=NS)


def _deg_call(dst3, zeros_np):
    @functools.partial(
        pl.kernel,
        out_type=jax.ShapeDtypeStruct((NC, NP), jnp.float32),
        mesh=_mesh(),
        scratch_types=[
            pltpu.VMEM((NCHUNK, CH), jnp.int32),
            pltpu.VMEM((CH,), jnp.float32),
            pltpu.VMEM_SHARED((NP,), jnp.float32),
        ],
    )
    def deg_k(dst_hbm, z_hbm, out_hbm, idx_v, ones_v, tab_sh):
        c = lax.axis_index("c")
        s = lax.axis_index("s")
        wid = s * NC + c
        for k in range(CH // 16):
            ones_v[pl.ds(k * 16, 16)] = jnp.ones((16,), jnp.float32)
        pltpu.sync_copy(z_hbm.at[pl.ds(s * SLC, SLC)],
                        tab_sh.at[pl.ds(s * SLC, SLC)])
        pltpu.sync_copy(dst_hbm.at[wid], idx_v)
        plsc.subcore_barrier()

        def body(j, carry):
            pltpu.sync_copy(ones_v, tab_sh.at[idx_v.at[j]], add=True)
            return carry

        lax.fori_loop(0, NCHUNK, body, 0)
        plsc.subcore_barrier()
        pltpu.sync_copy(tab_sh.at[pl.ds(s * SLC, SLC)],
                        out_hbm.at[c, pl.ds(s * SLC, SLC)])

    return deg_k(dst3, zeros_np)


def _agg_call(u, src3, dst3, zeros_tab):
    width = u.shape[1]

    @functools.partial(
        pl.kernel,
        out_type=jax.ShapeDtypeStruct((NC, NPAD, width), jnp.float32),
        mesh=_mesh(),
        scratch_types=[
            pltpu.VMEM((NCHUNK, CH), jnp.int32),
            pltpu.VMEM((NCHUNK, CH), jnp.int32),
            pltpu.VMEM((CH, width), jnp.float32),
            pltpu.VMEM_SHARED((NPAD, width), jnp.float32),
            pltpu.SemaphoreType.DMA,
        ],
    )
    def agg_k(u_hbm, src_hbm, dst_hbm, z_hbm, out_hbm,
              si_v, di_v, gb_v, acc_sh, sem):
        c = lax.axis_index("c")
        s = lax.axis_index("s")
        wid = s * NC + c
        pltpu.sync_copy(z_hbm.at[pl.ds(s * RPS, RPS)],
                        acc_sh.at[pl.ds(s * RPS, RPS)])
        pltpu.sync_copy(src_hbm.at[wid], si_v)
        pltpu.sync_copy(dst_hbm.at[wid], di_v)
        plsc.subcore_barrier()

        def body(j, carry):
            pltpu.async_copy(u_hbm.at[si_v.at[j]], gb_v, sem).wait()
            pltpu.sync_copy(gb_v, acc_sh.at[di_v.at[j]], add=True)
            return carry

        lax.fori_loop(0, NCHUNK, body, 0)
        plsc.subcore_barrier()
        pltpu.sync_copy(acc_sh.at[pl.ds(s * RPS, RPS)],
                        out_hbm.at[c, pl.ds(s * RPS, RPS)])

    return agg_k(u, src3, dst3, zeros_tab)


def _decode_call(z, pr3, pc3, nr3, nc3):
    D = z.shape[1]

    @functools.partial(
        pl.kernel,
        out_type=(jax.ShapeDtypeStruct((E,), jnp.float32),
                  jax.ShapeDtypeStruct((E,), jnp.float32)),
        compiler_params=pltpu.CompilerParams(needs_layout_passes=False),
        mesh=_mesh(),
        scratch_types=[
            pltpu.VMEM((NCHUNK, CH), jnp.int32),
            pltpu.VMEM((NCHUNK, CH), jnp.int32),
            pltpu.VMEM((CH, D), jnp.float32),
            pltpu.VMEM((CH, D), jnp.float32),
            pltpu.VMEM((CH,), jnp.float32),
            pltpu.SemaphoreType.DMA,
            pltpu.SemaphoreType.DMA,
        ],
    )
    def dec_k(z_hbm, pr_h, pc_h, nr_h, nc_h, po_h, no_h,
              ia_v, ib_v, a_v, b_v, sc_v, sa, sb):
        c = lax.axis_index("c")
        s = lax.axis_index("s")
        wid = s * NC + c

        def do_list(r_h, col_h, o_h):
            pltpu.sync_copy(r_h.at[wid], ia_v)
            pltpu.sync_copy(col_h.at[wid], ib_v)

            def body(j, carry):
                ca = pltpu.async_copy(z_hbm.at[ia_v.at[j]], a_v, sa)
                cb = pltpu.async_copy(z_hbm.at[ib_v.at[j]], b_v, sb)
                ca.wait()
                cb.wait()
                lane = lax.iota(jnp.int32, 16)
                for g in range(CH // 16):
                    svec = jnp.zeros((16,), jnp.float32)
                    for l in range(16):
                        e = g * 16 + l
                        part = a_v[e, pl.ds(0, 16)] * b_v[e, pl.ds(0, 16)]
                        for q in range(1, OUT_C // 16):
                            part = part + (a_v[e, pl.ds(q * 16, 16)] *
                                           b_v[e, pl.ds(q * 16, 16)])
                        tot = jnp.sum(part)
                        svec = jnp.where(lane == l, tot, svec)
                    sc_v[pl.ds(g * 16, 16)] = svec
                pltpu.sync_copy(sc_v, o_h.at[pl.ds(wid * EPW + j * CH, CH)])
                return carry

            lax.fori_loop(0, NCHUNK, body, 0)

        do_list(pr_h, pc_h, po_h)
        do_list(nr_h, nc_h, no_h)

    return dec_k(z, pr3, pc3, nr3, nc3)


def _dis_rows(dg_ref):
    deg = dg_ref[:, 0] + dg_ref[:, 1] + 1.0
    return lax.rsqrt(jnp.maximum(deg, 1e-12))


def _matmul(a, w_ref):
    return lax.dot_general(a, w_ref[...], (((1,), (0,)), ((), ())),
                           precision=lax.Precision.HIGHEST,
                           preferred_element_type=jnp.float32)


_BR = 2048


def _pre_call(degp, x, w1):
    def body(dg_ref, x_ref, w_ref, o_ref):
        dis = _dis_rows(dg_ref)[:, None]
        o_ref[...] = dis * _matmul(x_ref[...], w_ref)

    return pl.pallas_call(
        body,
        grid=(NPAD // _BR,),
        in_specs=[
            pl.BlockSpec((_BR, NC), lambda i: (i, 0)),
            pl.BlockSpec((_BR, IN_C), lambda i: (i, 0)),
            pl.BlockSpec((IN_C, HID), lambda i: (0, 0)),
        ],
        out_specs=pl.BlockSpec((_BR, HID), lambda i: (i, 0)),
        out_shape=jax.ShapeDtypeStruct((NPAD, HID), jnp.float32),
    )(degp, x, w1)


def _mid_call(degp, sp, u_prev, b, w_next):
    width = sp.shape[2]
    h = w_next.shape[1]

    def body(dg_ref, sp_ref, u_ref, b_ref, w_ref, o_ref):
        dis = _dis_rows(dg_ref)[:, None]
        stot = sp_ref[0] + sp_ref[1] + u_ref[...]
        zz = jnp.maximum(stot * dis + b_ref[...], 0.0)
        o_ref[...] = dis * _matmul(zz, w_ref)

    return pl.pallas_call(
        body,
        grid=(NPAD // _BR,),
        in_specs=[
            pl.BlockSpec((_BR, NC), lambda i: (i, 0)),
            pl.BlockSpec((NC, _BR, width), lambda i: (0, i, 0)),
            pl.BlockSpec((_BR, width), lambda i: (i, 0)),
            pl.BlockSpec((1, width), lambda i: (0, 0)),
            pl.BlockSpec((width, h), lambda i: (0, 0)),
        ],
        out_specs=pl.BlockSpec((_BR, h), lambda i: (i, 0)),
        out_shape=jax.ShapeDtypeStruct((NPAD, h), jnp.float32),
    )(degp, sp, u_prev, b, w_next)


def _fin_call(degp, sp, u_prev, b):
    width = sp.shape[2]

    def body(dg_ref, sp_ref, u_ref, b_ref, o_ref):
        dis = _dis_rows(dg_ref)[:, None]
        stot = sp_ref[0] + sp_ref[1] + u_ref[...]
        o_ref[...] = stot * dis + b_ref[...]

    return pl.pallas_call(
        body,
        grid=(NPAD // _BR,),
        in_specs=[
            pl.BlockSpec((_BR, NC), lambda i: (i, 0)),
            pl.BlockSpec((NC, _BR, width), lambda i: (0, i, 0)),
            pl.BlockSpec((_BR, width), lambda i: (i, 0)),
            pl.BlockSpec((1, width), lambda i: (0, 0)),
        ],
        out_specs=pl.BlockSpec((_BR, width), lambda i: (i, 0)),
        out_shape=jax.ShapeDtypeStruct((NPAD, width), jnp.float32),
    )(degp, sp, u_prev, b)


def kernel(x, pos_edge_index, neg_edge_index, W1, b1, W2, b2, W3, b3):
    src3 = pos_edge_index[0].reshape(NW, NCHUNK, CH)
    dst3 = pos_edge_index[1].reshape(NW, NCHUNK, CH)
    nr3 = neg_edge_index[0].reshape(NW, NCHUNK, CH)
    nc3 = neg_edge_index[1].reshape(NW, NCHUNK, CH)

    zeros_np = jnp.zeros((NP,), jnp.float32)
    zeros128 = jnp.zeros((NPAD, HID), jnp.float32)
    xp = jnp.pad(x, ((0, NPAD - N), (0, 0)))
    w3p = jnp.pad(W3, ((0, 0), (0, HID - OUT_C)))
    b3p = jnp.pad(b3, (0, HID - OUT_C))

    degp = _deg_call(dst3, zeros_np).T
    u1 = _pre_call(degp, xp, W1)
    s1 = _agg_call(u1, src3, dst3, zeros128)
    u2 = _mid_call(degp, s1, u1, b1.reshape(1, HID), W2)
    s2 = _agg_call(u2, src3, dst3, zeros128)
    u3 = _mid_call(degp, s2, u2, b2.reshape(1, HID), w3p)
    s3 = _agg_call(u3, src3, dst3, zeros128)
    z3 = _fin_call(degp, s3, u3, b3p.reshape(1, HID))
    pos_score, neg_score = _decode_call(z3, src3, dst3, nr3, nc3)
    return (pos_score, neg_score)
